# baseline (speedup 1.0000x reference)
import functools
import jax
import jax.numpy as jnp
from jax import lax
from jax.experimental import pallas as pl
from jax.experimental.pallas import tpu as pltpu
from jax.experimental.pallas import tpu_sc as plsc

_L = 1024
_UR = 50
_URL = _UR * _L
_UPS = 1000 // _UR
_UNITS = 26 * _UPS
_NW = 32
_MAXT = (_UNITS + _NW - 1) // _NW

_mesh = plsc.VectorSubcoreMesh(core_axis_name="c", subcore_axis_name="s")


@functools.partial(
    pl.kernel,
    out_type=jax.ShapeDtypeStruct((26000 * _L,), jnp.float32),
    mesh=_mesh,
    scratch_types=[
        pltpu.VMEM((2 * _URL,), jnp.float32),
        pltpu.VMEM((_L,), jnp.int32),
        pltpu.SemaphoreType.DMA,
    ],
)
def _sc_one_hot(xt_hbm, out_hbm, buf, xcol, sem):
    w = lax.axis_index("s") * 2 + lax.axis_index("c")

    def step(t, carry):
        u = w + _NW * t

        @pl.when(u < _UNITS)
        def _unit():
            boff = (t % 2) * _URL
            j = u // _UPS
            c0 = (u % _UPS) * _UR
            pltpu.sync_copy(xt_hbm.at[pl.ds(j * _L, _L)], xcol)

            def row_body(local, rc):
                c = c0 + local
                for i in range(_L // 16):
                    xv = xcol[pl.ds(i * 16, 16)]
                    v = jnp.where(xv == c, jnp.float32(1.0), jnp.float32(0.0))
                    buf[pl.ds(boff + local * _L + i * 16, 16)] = v
                return rc

            lax.fori_loop(0, _UR, row_body, 0)

            @pl.when(t >= 1)
            def _wait_prev():
                pltpu.make_async_copy(
                    buf.at[pl.ds(0, _URL)],
                    out_hbm.at[pl.ds(0, _URL)], sem).wait()

            pltpu.async_copy(
                buf.at[pl.ds(boff, _URL)],
                out_hbm.at[pl.ds((j * 1000 + c0) * _L, _URL)], sem)

        return carry

    lax.fori_loop(0, _MAXT, step, 0)
    pltpu.make_async_copy(
        buf.at[pl.ds(0, _URL)], out_hbm.at[pl.ds(0, _URL)], sem).wait()


def kernel(x):
    xt = x.astype(jnp.int32).T.reshape(26 * _L)
    y = _sc_one_hot(xt)
    return jnp.transpose(y.reshape(26, 1000, _L), (2, 0, 1))


# speedup vs baseline: 11.9769x; 11.9769x over previous
import jax
import jax.numpy as jnp
from jax.experimental import pallas as pl

_NUM_CLASSES = 1000
_N = 1024


def _body(x_ref, o_ref):
    j = pl.program_id(0)
    xv = x_ref[pl.ds(j, 1), :].reshape(1, 1, _N)
    row = jax.lax.broadcasted_iota(jnp.int32, (1, _NUM_CLASSES, _N), 1)
    o_ref[...] = (row == xv).astype(jnp.float32)


def kernel(x):
    xt = x.astype(jnp.int32).T
    yt = pl.pallas_call(
        _body,
        grid=(26,),
        in_specs=[pl.BlockSpec((26, _N), lambda j: (0, 0))],
        out_specs=pl.BlockSpec((1, _NUM_CLASSES, _N), lambda j: (j, 0, 0)),
        out_shape=jax.ShapeDtypeStruct((26, _NUM_CLASSES, _N), jnp.float32),
    )(xt)
    return jnp.transpose(yt, (2, 0, 1))
